# baseline (speedup 1.0000x reference)
import functools

import jax
import jax.numpy as jnp
from jax import lax
from jax.experimental import pallas as pl
from jax.experimental.pallas import tpu as pltpu
from jax.experimental.pallas import tpu_sc as plsc

K = 8192
L = 16
NS = 16
PER_SUB = K // NS
VECS_IN = PER_SUB // L


def _lane_shuffle(v, perm):
    return v.at[perm].get(mode="promise_in_bounds")


def _merge_pair(av, ai, bv, bi):
    better = (bv < av) | ((bv == av) & (bi < ai))
    return jnp.where(better, bv, av), jnp.where(better, bi, ai)


def _butterfly_min_pair(vmin, vidx, iota):
    for sh in (1, 2, 4, 8):
        perm = iota ^ sh
        pv = _lane_shuffle(vmin, perm)
        pi = _lane_shuffle(vidx, perm)
        vmin, vidx = _merge_pair(vmin, vidx, pv, pi)
    return vmin, vidx


def _body(x_hbm, out_hbm, xv, pub, spub, gpub, zv, onev, sem):
    s = lax.axis_index("s")
    iota = lax.iota(jnp.int32, L)
    base = s * PER_SUB

    pltpu.sync_copy(x_hbm.at[pl.ds(base, PER_SUB)], xv)

    zero = jnp.full((L,), 0.0, jnp.float32)
    for j in range(VECS_IN):
        zv[pl.ds(j * L, L)] = zero
    zdma = pltpu.async_copy(zv, out_hbm.at[pl.ds(base, PER_SUB)], sem)

    vmin = jnp.full((L,), jnp.inf, jnp.float32)
    vidx = jnp.zeros((L,), jnp.int32)
    for j in range(VECS_IN):
        xj = xv[pl.ds(j * L, L)]
        ij = iota + (base + j * L)
        vidx = jnp.where(xj < vmin, ij, vidx)
        vmin = jnp.minimum(vmin, xj)

    lmin_v, lidx_v = _butterfly_min_pair(vmin, vidx, iota)

    pub[pl.ds(0, L)] = lmin_v
    pub[pl.ds(L, L)] = lidx_v.astype(jnp.float32)
    pltpu.sync_copy(pub, spub.at[pl.ds(s * 2 * L, 2 * L)])
    zdma.wait()
    plsc.subcore_barrier()

    pltpu.sync_copy(spub, gpub)
    gmin_v = gpub[pl.ds(0, L)]
    gidx_f = gpub[pl.ds(L, L)]
    for r in range(1, NS):
        rv = gpub[pl.ds(r * 2 * L, L)]
        ri = gpub[pl.ds(r * 2 * L + L, L)]
        gmin_v, gidx_f = _merge_pair(gmin_v, gidx_f, rv, ri)
    gidx_v = gidx_f.astype(jnp.int32)
    onev[...] = jnp.full((L,), 1.0, jnp.float32)
    pltpu.async_copy(onev, out_hbm.at[gidx_v], sem).wait()


@functools.partial(
    pl.kernel,
    out_type=jax.ShapeDtypeStruct((K,), jnp.float32),
    mesh=plsc.VectorSubcoreMesh(core_axis_name="c", subcore_axis_name="s",
                                num_cores=1),
    scratch_types=[
        pltpu.VMEM((PER_SUB,), jnp.float32),
        pltpu.VMEM((2 * L,), jnp.float32),
        pltpu.VMEM_SHARED((NS * 2 * L,), jnp.float32),
        pltpu.VMEM((NS * 2 * L,), jnp.float32),
        pltpu.VMEM((PER_SUB,), jnp.float32),
        pltpu.VMEM((L,), jnp.float32),
        pltpu.SemaphoreType.DMA,
    ],
)
def _stargmin_sc(x_hbm, out_hbm, *scratch):
    _body(x_hbm, out_hbm, *scratch)


def kernel(x):
    return _stargmin_sc(x.reshape(K)).reshape(1, K)


# speedup vs baseline: 3.0311x; 3.0311x over previous
import functools

import jax
import jax.numpy as jnp
from jax import lax
from jax.experimental import pallas as pl
from jax.experimental.pallas import tpu as pltpu
from jax.experimental.pallas import tpu_sc as plsc

K = 8192
L = 16
NS = 16
PER_SUB = K // NS
VECS_IN = PER_SUB // L


def _lane_shuffle(v, perm):
    return v.at[perm].get(mode="promise_in_bounds")


def _merge_pair(av, ai, bv, bi):
    better = (bv < av) | ((bv == av) & (bi < ai))
    return jnp.where(better, bv, av), jnp.where(better, bi, ai)


def _butterfly_min_pair(vmin, vidx, iota):
    for sh in (1, 2, 4, 8):
        perm = iota ^ sh
        pv = _lane_shuffle(vmin, perm)
        pi = _lane_shuffle(vidx, perm)
        vmin, vidx = _merge_pair(vmin, vidx, pv, pi)
    return vmin, vidx


def _body(x_hbm, out_hbm, xv, pub, spub, gpub, ov):
    s = lax.axis_index("s")
    iota = lax.iota(jnp.int32, L)
    base = s * PER_SUB

    pltpu.sync_copy(x_hbm.at[pl.ds(base, PER_SUB)], xv)

    vmin = jnp.full((L,), jnp.inf, jnp.float32)
    vidx = jnp.zeros((L,), jnp.int32)
    for j in range(VECS_IN):
        xj = xv[pl.ds(j * L, L)]
        ij = iota + (base + j * L)
        vidx = jnp.where(xj < vmin, ij, vidx)
        vmin = jnp.minimum(vmin, xj)

    lmin_v, lidx_v = _butterfly_min_pair(vmin, vidx, iota)

    pub[pl.ds(0, L)] = lmin_v
    pub[pl.ds(L, L)] = lidx_v.astype(jnp.float32)
    pltpu.sync_copy(pub, spub.at[pl.ds(s * 2 * L, 2 * L)])
    plsc.subcore_barrier()

    pltpu.sync_copy(spub, gpub)
    gmin_v = gpub[pl.ds(0, L)]
    gidx_f = gpub[pl.ds(L, L)]
    for r in range(1, NS):
        rv = gpub[pl.ds(r * 2 * L, L)]
        ri = gpub[pl.ds(r * 2 * L + L, L)]
        gmin_v, gidx_f = _merge_pair(gmin_v, gidx_f, rv, ri)
    gidx_v = gidx_f.astype(jnp.int32)

    one = jnp.full((L,), 1.0, jnp.float32)
    zero = jnp.full((L,), 0.0, jnp.float32)
    for j in range(VECS_IN):
        pos = iota + (base + j * L)
        ov[pl.ds(j * L, L)] = jnp.where(pos == gidx_v, one, zero)
    pltpu.sync_copy(ov, out_hbm.at[pl.ds(base, PER_SUB)])


@functools.partial(
    pl.kernel,
    out_type=jax.ShapeDtypeStruct((K,), jnp.float32),
    mesh=plsc.VectorSubcoreMesh(core_axis_name="c", subcore_axis_name="s",
                                num_cores=1),
    scratch_types=[
        pltpu.VMEM((PER_SUB,), jnp.float32),
        pltpu.VMEM((2 * L,), jnp.float32),
        pltpu.VMEM_SHARED((NS * 2 * L,), jnp.float32),
        pltpu.VMEM((NS * 2 * L,), jnp.float32),
        pltpu.VMEM((PER_SUB,), jnp.float32),
    ],
)
def _stargmin_sc(x_hbm, out_hbm, *scratch):
    _body(x_hbm, out_hbm, *scratch)


def kernel(x):
    return _stargmin_sc(x.reshape(K)).reshape(1, K)


# speedup vs baseline: 3.0352x; 1.0013x over previous
import functools

import jax
import jax.numpy as jnp
from jax import lax
from jax.experimental import pallas as pl
from jax.experimental.pallas import tpu as pltpu
from jax.experimental.pallas import tpu_sc as plsc

K = 8192
L = 16
NS = 16
PER_SUB = K // NS
VECS_IN = PER_SUB // L


def _lane_shuffle(v, perm):
    return v.at[perm].get(mode="promise_in_bounds")


def _merge_pair(av, ai, bv, bi):
    better = (bv < av) | ((bv == av) & (bi < ai))
    return jnp.where(better, bv, av), jnp.where(better, bi, ai)


def _butterfly_min_pair(vmin, vidx, iota):
    for sh in (1, 2, 4, 8):
        perm = iota ^ sh
        pv = _lane_shuffle(vmin, perm)
        pi = _lane_shuffle(vidx, perm)
        vmin, vidx = _merge_pair(vmin, vidx, pv, pi)
    return vmin, vidx


def _body(x_hbm, out_hbm, xv, pub, spub, gpub, ov, sem):
    s = lax.axis_index("s")
    iota = lax.iota(jnp.int32, L)
    base = s * PER_SUB

    pltpu.sync_copy(x_hbm.at[pl.ds(base, PER_SUB)], xv)

    zero = jnp.full((L,), 0.0, jnp.float32)
    for j in range(VECS_IN):
        ov[pl.ds(j * L, L)] = zero
    zdma = pltpu.async_copy(ov, out_hbm.at[pl.ds(base, PER_SUB)], sem)

    vmin = jnp.full((L,), jnp.inf, jnp.float32)
    vidx = jnp.zeros((L,), jnp.int32)
    for j in range(VECS_IN):
        xj = xv[pl.ds(j * L, L)]
        ij = iota + (base + j * L)
        vidx = jnp.where(xj < vmin, ij, vidx)
        vmin = jnp.minimum(vmin, xj)

    lmin_v, lidx_v = _butterfly_min_pair(vmin, vidx, iota)

    pub[pl.ds(0, L)] = lmin_v
    pub[pl.ds(L, L)] = lidx_v.astype(jnp.float32)
    pltpu.sync_copy(pub, spub.at[pl.ds(s * 2 * L, 2 * L)])
    zdma.wait()
    plsc.subcore_barrier()

    pltpu.sync_copy(spub, gpub)
    gmin_v = gpub[pl.ds(0, L)]
    gidx_f = gpub[pl.ds(L, L)]
    for r in range(1, NS):
        rv = gpub[pl.ds(r * 2 * L, L)]
        ri = gpub[pl.ds(r * 2 * L + L, L)]
        gmin_v, gidx_f = _merge_pair(gmin_v, gidx_f, rv, ri)
    gidx_v = gidx_f.astype(jnp.int32)
    gidx = gidx_v[0]

    @pl.when(gidx // PER_SUB == s)
    def _():
        one = jnp.full((L,), 1.0, jnp.float32)
        zero = jnp.full((L,), 0.0, jnp.float32)
        ov[pl.ds(0, L)] = jnp.where(iota == (gidx_v & (L - 1)), one, zero)
        pltpu.sync_copy(ov.at[pl.ds(0, L)],
                        out_hbm.at[pl.ds((gidx // L) * L, L)])


@functools.partial(
    pl.kernel,
    out_type=jax.ShapeDtypeStruct((K,), jnp.float32),
    mesh=plsc.VectorSubcoreMesh(core_axis_name="c", subcore_axis_name="s",
                                num_cores=1),
    scratch_types=[
        pltpu.VMEM((PER_SUB,), jnp.float32),
        pltpu.VMEM((2 * L,), jnp.float32),
        pltpu.VMEM_SHARED((NS * 2 * L,), jnp.float32),
        pltpu.VMEM((NS * 2 * L,), jnp.float32),
        pltpu.VMEM((PER_SUB,), jnp.float32),
        pltpu.SemaphoreType.DMA,
    ],
)
def _stargmin_sc(x_hbm, out_hbm, *scratch):
    _body(x_hbm, out_hbm, *scratch)


def kernel(x):
    return _stargmin_sc(x.reshape(K)).reshape(1, K)
